# baseline (speedup 1.0000x reference)
import functools

import jax
import jax.numpy as jnp
from jax import lax
from jax.experimental import pallas as pl
from jax.experimental.pallas import tpu as pltpu
from jax.experimental.pallas import tpu_sc as plsc

D = 8192
D2 = D // 2
B_TOT = 16 * 512
NW = 32
RPW = B_TOT // NW
SUB = 2 * RPW
CH = 8
NCH = SUB // CH


def _gather_body(table_hbm, idx2_hbm, out_hbm, idx2_v, buf0, buf1,
                 g0, g1, s0, s1):
    wid = lax.axis_index("s") * 2 + lax.axis_index("c")
    pltpu.sync_copy(idx2_hbm.at[pl.ds(wid * SUB, SUB)], idx2_v)

    bufs = (buf0, buf1)
    gs = (g0, g1)
    ss = (s0, s1)
    ob = wid * SUB

    def g_copy(c, s):
        return pltpu.make_async_copy(
            table_hbm.at[idx2_v.at[pl.ds(c * CH, CH)]], bufs[s], gs[s])

    def w_copy(c, s):
        return pltpu.make_async_copy(
            bufs[s], out_hbm.at[pl.ds(ob + c * CH, CH)], ss[s])

    g_copy(0, 0).start()
    g_copy(1, 1).start()

    def group(g, carry):
        for b in range(2):
            c = 2 * g + b
            g_copy(c, b).wait()
            w_copy(c, b).start()
            w_copy(c, b).wait()
            g_copy(c + 2, b).start()
        return carry

    lax.fori_loop(0, (NCH - 2) // 2, group, 0)

    for b in range(2):
        c = NCH - 2 + b
        g_copy(c, b).wait()
        w_copy(c, b).start()
    for b in range(2):
        w_copy(NCH - 2 + b, b).wait()


@jax.jit
def _gather(table2, idx2):
    mesh = plsc.VectorSubcoreMesh(core_axis_name="c", subcore_axis_name="s")
    k = functools.partial(
        pl.kernel,
        out_type=jax.ShapeDtypeStruct((2 * B_TOT, D2), jnp.float32),
        mesh=mesh,
        scratch_types=[
            pltpu.VMEM((SUB,), jnp.int32),
            pltpu.VMEM((CH, D2), jnp.float32),
            pltpu.VMEM((CH, D2), jnp.float32),
            pltpu.SemaphoreType.DMA,
            pltpu.SemaphoreType.DMA,
            pltpu.SemaphoreType.DMA,
            pltpu.SemaphoreType.DMA,
        ],
    )(_gather_body)
    return k(table2, idx2)


def kernel(idx, targets, table):
    del targets
    idx_flat = idx.reshape(-1).astype(jnp.int32)
    idx2 = (2 * idx_flat[:, None] + jnp.arange(2, dtype=jnp.int32)).reshape(-1)
    table2 = table.reshape(2 * table.shape[0], D2)
    out = _gather(table2, idx2)
    return out.reshape(idx.shape[0], idx.shape[1], D)


# speedup vs baseline: 3.6563x; 3.6563x over previous
import functools

import jax
import jax.numpy as jnp
from jax import lax
from jax.experimental import pallas as pl
from jax.experimental.pallas import tpu as pltpu
from jax.experimental.pallas import tpu_sc as plsc

D = 8192
B_TOT = 16 * 512
NW = 32
RPW = B_TOT // NW
CH = 4
NCH = RPW // CH


def _gather_body(table_hbm, idx_hbm, out_hbm, idx_v, buf0, buf1,
                 g0, g1, s0, s1):
    wid = lax.axis_index("s") * 2 + lax.axis_index("c")
    base = wid * RPW
    pltpu.sync_copy(idx_hbm.at[pl.ds(wid * NCH, NCH)], idx_v)

    bufs = (buf0, buf1)
    gs = (g0, g1)
    ss = (s0, s1)

    def g_copy(c, s):
        return pltpu.make_async_copy(
            table_hbm.at[idx_v.at[c]], bufs[s], gs[s])

    def w_copy(c, s):
        return pltpu.make_async_copy(
            bufs[s], out_hbm.at[pl.ds(base + c * CH, CH)], ss[s])

    g_copy(0, 0).start()
    g_copy(1, 1).start()

    def group(g, carry):
        for b in range(2):
            c = 2 * g + b
            g_copy(c, b).wait()
            w_copy(c, b).start()
            w_copy(c, b).wait()
            g_copy(c + 2, b).start()
        return carry

    lax.fori_loop(0, (NCH - 2) // 2, group, 0)

    for b in range(2):
        c = NCH - 2 + b
        g_copy(c, b).wait()
        w_copy(c, b).start()
    for b in range(2):
        w_copy(NCH - 2 + b, b).wait()


@jax.jit
def _gather(table, idx_flat):
    mesh = plsc.VectorSubcoreMesh(core_axis_name="c", subcore_axis_name="s")
    k = functools.partial(
        pl.kernel,
        out_type=jax.ShapeDtypeStruct((B_TOT, D), jnp.float32),
        mesh=mesh,
        scratch_types=[
            pltpu.VMEM((NCH, CH), jnp.int32),
            pltpu.VMEM((CH, D), jnp.float32),
            pltpu.VMEM((CH, D), jnp.float32),
            pltpu.SemaphoreType.DMA,
            pltpu.SemaphoreType.DMA,
            pltpu.SemaphoreType.DMA,
            pltpu.SemaphoreType.DMA,
        ],
    )(_gather_body)
    return k(table, idx_flat)


def kernel(idx, targets, table):
    del targets
    idx2d = idx.reshape(B_TOT // CH, CH).astype(jnp.int32)
    out = _gather(table, idx2d)
    return out.reshape(idx.shape[0], idx.shape[1], D)


# speedup vs baseline: 3.6813x; 1.0068x over previous
import functools

import jax
import jax.numpy as jnp
from jax import lax
from jax.experimental import pallas as pl
from jax.experimental.pallas import tpu as pltpu
from jax.experimental.pallas import tpu_sc as plsc

D = 8192
B_TOT = 16 * 512
NW = 32
RPW = B_TOT // NW
CH = 4
NCH = RPW // CH


def _gather_body(table_hbm, idx_hbm, out_hbm, idx_v, buf0, buf1, buf2,
                 g0, g1, g2, s0, s1, s2):
    wid = lax.axis_index("s") * 2 + lax.axis_index("c")
    base = wid * RPW
    pltpu.sync_copy(idx_hbm.at[pl.ds(wid * NCH, NCH)], idx_v)

    bufs = (buf0, buf1, buf2)
    gs = (g0, g1, g2)
    ss = (s0, s1, s2)

    def g_copy(c, s):
        return pltpu.make_async_copy(
            table_hbm.at[idx_v.at[c]], bufs[s], gs[s])

    def w_copy(c, s):
        return pltpu.make_async_copy(
            bufs[s], out_hbm.at[pl.ds(base + c * CH, CH)], ss[s])

    g_copy(0, 0).start()
    g_copy(1, 1).start()
    g_copy(2, 2).start()
    g_copy(0, 0).wait()
    w_copy(0, 0).start()

    def group(g, carry):
        for b in range(3):
            i = 3 * g_off + 3 * g + b
            w_copy(i - 3, b).wait()
            g_copy(i, b).start()
            g_copy(i - 2, (b + 1) % 3).wait()
            w_copy(i - 2, (b + 1) % 3).start()
        return carry

    g_off = 1
    NGRP = (NCH - 3 - 1) // 3
    lax.fori_loop(0, NGRP, group, 0)

    for i in range(3 + 3 * NGRP, NCH):
        b = i % 3
        w_copy(i - 3, b).wait()
        g_copy(i, b).start()
        g_copy(i - 2, (i - 2) % 3).wait()
        w_copy(i - 2, (i - 2) % 3).start()

    for c in (NCH - 2, NCH - 1):
        g_copy(c, c % 3).wait()
        w_copy(c, c % 3).start()
    for c in (NCH - 3, NCH - 2, NCH - 1):
        w_copy(c, c % 3).wait()


@jax.jit
def _gather(table, idx_flat):
    mesh = plsc.VectorSubcoreMesh(core_axis_name="c", subcore_axis_name="s")
    k = functools.partial(
        pl.kernel,
        out_type=jax.ShapeDtypeStruct((B_TOT, D), jnp.float32),
        mesh=mesh,
        scratch_types=[
            pltpu.VMEM((NCH, CH), jnp.int32),
            pltpu.VMEM((CH, D), jnp.float32),
            pltpu.VMEM((CH, D), jnp.float32),
            pltpu.VMEM((CH, D), jnp.float32),
            pltpu.SemaphoreType.DMA,
            pltpu.SemaphoreType.DMA,
            pltpu.SemaphoreType.DMA,
            pltpu.SemaphoreType.DMA,
            pltpu.SemaphoreType.DMA,
            pltpu.SemaphoreType.DMA,
        ],
    )(_gather_body)
    return k(table, idx_flat)


def kernel(idx, targets, table):
    del targets
    idx2d = idx.reshape(B_TOT // CH, CH).astype(jnp.int32)
    out = _gather(table, idx2d)
    return out.reshape(idx.shape[0], idx.shape[1], D)
